# baseline (speedup 1.0000x reference)
import jax
import jax.numpy as jnp
from jax.experimental import pallas as pl
from jax.experimental.pallas import tpu as pltpu

_N = 8192
_BM = 4096


def _chamfer_body(a_ref, bt_ref, out_ref, d1_acc, s0_acc):
    i = pl.program_id(0)
    ni = pl.num_programs(0)

    a = a_ref[...]
    bt = bt_ref[...]
    f32, bf16 = jnp.float32, jnp.bfloat16

    a2 = -2.0 * a
    ahi = a2.astype(bf16)
    alo = (a2 - ahi.astype(f32)).astype(bf16)
    n0 = jnp.sum(a * a, axis=1, keepdims=True)
    n0h = n0.astype(bf16)
    r = n0 - n0h.astype(f32)
    n0m = r.astype(bf16)
    n0l = (r - n0m.astype(f32)).astype(bf16)
    ones_a = jnp.ones((_BM, 3), bf16)
    a15 = jnp.concatenate(
        [ahi, ahi, alo, n0h, n0m, n0l, ones_a], axis=1)

    bhi = bt.astype(bf16)
    blo = (bt - bhi.astype(f32)).astype(bf16)
    n1 = jnp.sum(bt * bt, axis=0, keepdims=True)
    n1h = n1.astype(bf16)
    s = n1 - n1h.astype(f32)
    n1m = s.astype(bf16)
    n1l = (s - n1m.astype(f32)).astype(bf16)
    ones_b = jnp.ones((3, _N), bf16)
    b15 = jnp.concatenate(
        [bhi, blo, bhi, ones_b, n1h, n1m, n1l], axis=0)

    d2 = jnp.dot(a15, b15, preferred_element_type=f32)

    row_min = jnp.min(d2, axis=1)
    col_min = jnp.min(d2, axis=0, keepdims=True)

    @pl.when(i == 0)
    def _init():
        d1_acc[...] = col_min
        s0_acc[0, 0] = 0.0

    @pl.when(i != 0)
    def _accum():
        d1_acc[...] = jnp.minimum(d1_acc[...], col_min)

    s0_acc[0, 0] += jnp.sum(jnp.maximum(row_min, 0.0))

    @pl.when(i == ni - 1)
    def _finish():
        d1_sum = jnp.sum(jnp.maximum(d1_acc[...], 0.0))
        loss = (s0_acc[0, 0] + d1_sum) / float(_N)
        out_ref[...] = jnp.broadcast_to(loss, (1, 1))


def _chamfer(pc0, pc1t):
    ni = _N // _BM
    out = pl.pallas_call(
        _chamfer_body,
        grid=(ni,),
        in_specs=[
            pl.BlockSpec((_BM, 3), lambda i: (i, 0)),
            pl.BlockSpec((3, _N), lambda i: (0, 0)),
        ],
        out_specs=pl.BlockSpec((1, 1), lambda i: (0, 0)),
        out_shape=jax.ShapeDtypeStruct((1, 1), jnp.float32),
        scratch_shapes=[
            pltpu.VMEM((1, _N), jnp.float32),
            pltpu.SMEM((1, 1), jnp.float32),
        ],
        compiler_params=pltpu.CompilerParams(
            vmem_limit_bytes=128 * 1024 * 1024),
    )(pc0, pc1t)
    return out[0, 0]


@jax.jit
def kernel(input0, input1):
    return _chamfer(input0, input1.T)


# speedup vs baseline: 1.1359x; 1.1359x over previous
import jax
import jax.numpy as jnp
from jax.experimental import pallas as pl
from jax.experimental.pallas import tpu as pltpu

_N = 8192
_BM = 4096


def _chamfer_body(a_ref, bt_ref, out_ref, d1_acc, s0_acc):
    i = pl.program_id(0)
    ni = pl.num_programs(0)

    a = a_ref[...]
    bt = bt_ref[...]
    n0 = jnp.sum(a * a, axis=1, keepdims=True)
    n1 = jnp.sum(bt * bt, axis=0, keepdims=True)
    prod = jnp.dot(-2.0 * a, bt, preferred_element_type=jnp.float32)

    row_min = jnp.min(prod + n1, axis=1) + n0[:, 0]
    col_min = jnp.min(prod + n0, axis=0, keepdims=True)

    @pl.when(i == 0)
    def _init():
        d1_acc[...] = col_min
        s0_acc[0, 0] = 0.0

    @pl.when(i != 0)
    def _accum():
        d1_acc[...] = jnp.minimum(d1_acc[...], col_min)

    s0_acc[0, 0] += jnp.sum(jnp.maximum(row_min, 0.0))

    @pl.when(i == ni - 1)
    def _finish():
        n1_fin = jnp.sum(bt_ref[...] * bt_ref[...], axis=0, keepdims=True)
        d1_sum = jnp.sum(jnp.maximum(d1_acc[...] + n1_fin, 0.0))
        loss = (s0_acc[0, 0] + d1_sum) / float(_N)
        out_ref[...] = jnp.broadcast_to(loss, (1, 1))


def _chamfer(pc0, pc1t):
    ni = _N // _BM
    out = pl.pallas_call(
        _chamfer_body,
        grid=(ni,),
        in_specs=[
            pl.BlockSpec((_BM, 3), lambda i: (i, 0)),
            pl.BlockSpec((3, _N), lambda i: (0, 0)),
        ],
        out_specs=pl.BlockSpec((1, 1), lambda i: (0, 0)),
        out_shape=jax.ShapeDtypeStruct((1, 1), jnp.float32),
        scratch_shapes=[
            pltpu.VMEM((1, _N), jnp.float32),
            pltpu.SMEM((1, 1), jnp.float32),
        ],
        compiler_params=pltpu.CompilerParams(
            vmem_limit_bytes=128 * 1024 * 1024),
    )(pc0, pc1t)
    return out[0, 0]


@jax.jit
def kernel(input0, input1):
    return _chamfer(input0, input1.T)
